# baseline (speedup 1.0000x reference)
import jax
import jax.numpy as jnp
from jax import lax
from jax.experimental import pallas as pl
from jax.experimental.pallas import tpu as pltpu
from jax.experimental.pallas import tpu_sc as plsc

_B = 4096
_IMG_D = 3 * 64 * 64
_OUT_D = _IMG_D + 16 + 16 + 8

_NC, _NS = 2, 16
_NW = _NC * _NS
_BPW = _B // _NW


def _sc_gather_body(verb_hbm, noun_hbm, color_hbm, emb_v_hbm, emb_n_hbm, emb_c_hbm,
                    ev_out, en_out, ec_out,
                    vidx, nidx, cidx, rv, rn, rc, sem):
    wid = lax.axis_index("s") * _NC + lax.axis_index("c")
    base = wid * _BPW
    pltpu.sync_copy(verb_hbm.at[pl.ds(base, _BPW)], vidx)
    pltpu.sync_copy(noun_hbm.at[pl.ds(base, _BPW)], nidx)
    pltpu.sync_copy(color_hbm.at[pl.ds(base, _BPW)], cidx)
    descs = []
    for g in range(_BPW // 16):
        vv = vidx[pl.ds(g * 16, 16)]
        nv = nidx[pl.ds(g * 16, 16)]
        cv = cidx[pl.ds(g * 16, 16)]
        for i in range(16):
            r = g * 16 + i
            descs.append(pltpu.async_copy(emb_v_hbm.at[vv[i]], rv.at[r], sem))
            descs.append(pltpu.async_copy(emb_n_hbm.at[nv[i]], rn.at[r], sem))
            descs.append(pltpu.async_copy(emb_c_hbm.at[cv[i]], rc.at[r], sem))
    for d in descs:
        d.wait()
    pltpu.sync_copy(rv, ev_out.at[pl.ds(base, _BPW)])
    pltpu.sync_copy(rn, en_out.at[pl.ds(base, _BPW)])
    pltpu.sync_copy(rc, ec_out.at[pl.ds(base, _BPW)])


def _sc_gather(verb, noun, color, emb_v, emb_n, emb_c):
    mesh = plsc.VectorSubcoreMesh(core_axis_name="c", subcore_axis_name="s",
                                  num_cores=_NC, num_subcores=_NS)
    f = pl.kernel(
        _sc_gather_body,
        out_type=[jax.ShapeDtypeStruct((_B, 16), jnp.float32),
                  jax.ShapeDtypeStruct((_B, 16), jnp.float32),
                  jax.ShapeDtypeStruct((_B, 8), jnp.float32)],
        mesh=mesh,
        scratch_types=[pltpu.VMEM((_BPW,), jnp.int32),
                       pltpu.VMEM((_BPW,), jnp.int32),
                       pltpu.VMEM((_BPW,), jnp.int32),
                       pltpu.VMEM((_BPW, 16), jnp.float32),
                       pltpu.VMEM((_BPW, 16), jnp.float32),
                       pltpu.VMEM((_BPW, 8), jnp.float32),
                       pltpu.SemaphoreType.DMA],
    )
    return f(verb, noun, color, emb_v, emb_n, emb_c)


_SR = 64
_NSL = _B // _SR
_NI = 4
_NO = 4


def _img_body(img_hbm, out_hbm, *args):
    bufs_in = args[:_NI]
    bufs_out = args[_NI:_NI + _NO]
    isems = args[_NI + _NO:_NI + _NO + _NI]
    osems = args[_NI + _NO + _NI:]

    def in_desc(i, b):
        return pltpu.make_async_copy(img_hbm.at[pl.ds(i * _SR, _SR)],
                                     bufs_in[b], isems[b])

    def out_desc(i, b):
        return pltpu.make_async_copy(
            bufs_out[b],
            out_hbm.at[pl.ds(i * _SR, _SR), pl.ds(0, _IMG_D)],
            osems[b])

    for b in range(_NI):
        in_desc(b, b).start()

    def round_body(g, carry):
        for b in range(_NI):
            i = g * _NI + b
            in_desc(i, b).wait()

            @pl.when(g > 0)
            def _():
                out_desc(i - _NI, b).wait()

            bufs_out[b][...] = bufs_in[b][...].reshape(_SR, _IMG_D)
            out_desc(i, b).start()

            @pl.when(i + _NI < _NSL)
            def _():
                in_desc(i + _NI, b).start()
        return carry

    lax.fori_loop(0, _NSL // _NI, round_body, 0)
    for b in range(_NI):
        out_desc(_NSL - _NI + b, b).wait()


def _img_copy(img4d):
    return pl.pallas_call(
        _img_body,
        in_specs=[pl.BlockSpec(memory_space=pl.ANY)],
        out_specs=pl.BlockSpec(memory_space=pl.ANY),
        out_shape=jax.ShapeDtypeStruct((_B, _OUT_D), jnp.float32),
        scratch_shapes=([pltpu.VMEM((_SR, 3, 64, 64), jnp.float32)] * _NI
                        + [pltpu.VMEM((_SR, _IMG_D), jnp.float32)] * _NO
                        + [pltpu.SemaphoreType.DMA] * (_NI + _NO)),
    )(img4d)


def _memb_body(out_in, ev_ref, en_ref, ec_ref, out_ref):
    out_ref[:, 0:16] = ev_ref[...]
    out_ref[:, 16:32] = en_ref[...]
    out_ref[:, 32:40] = ec_ref[...]


def _memb_write(out1, ev, en, ec):
    return pl.pallas_call(
        _memb_body,
        grid=(1,),
        in_specs=[pl.BlockSpec(memory_space=pl.ANY),
                  pl.BlockSpec((_B, 16), lambda i: (0, 0)),
                  pl.BlockSpec((_B, 16), lambda i: (0, 0)),
                  pl.BlockSpec((_B, 8), lambda i: (0, 0))],
        out_specs=pl.BlockSpec((_B, 128), lambda i: (0, 96)),
        out_shape=jax.ShapeDtypeStruct((_B, _OUT_D), jnp.float32),
        input_output_aliases={0: 0},
    )(out1, ev, en, ec)


def kernel(image, verb, noun, color, emb_v, emb_n, emb_c):
    img4d = image.astype(jnp.float32)
    verb = verb.astype(jnp.int32)
    noun = noun.astype(jnp.int32)
    color = color.astype(jnp.int32)
    ev, en, ec = _sc_gather(verb, noun, color,
                            emb_v.astype(jnp.float32),
                            emb_n.astype(jnp.float32),
                            emb_c.astype(jnp.float32))
    out1 = _img_copy(img4d)
    return _memb_write(out1, ev, en, ec)


# speedup vs baseline: 2.4149x; 2.4149x over previous
import jax
import jax.numpy as jnp
from jax import lax
from jax.experimental import pallas as pl
from jax.experimental.pallas import tpu as pltpu
from jax.experimental.pallas import tpu_sc as plsc

_B = 4096
_IMG_D = 3 * 64 * 64
_DV, _DN, _DC = 16, 16, 8
_DM = _DV + _DN + _DC
_OUT_D = _IMG_D + _DM
_VV, _VN, _VC = 1000, 100000, 16

_NC, _NS = 2, 16
_NW = _NC * _NS
_BPW = _B // _NW
_L = 16


def _sc_gather_body(verb_hbm, noun_hbm, color_hbm, evf_hbm, enf_hbm, ecf_hbm,
                    memb_out,
                    vidx, nidx, cidx, iv, inn, ic, rv, rn, rc,
                    sem_v, sem_n, sem_c):
    wid = lax.axis_index("s") * _NC + lax.axis_index("c")
    base = wid * _BPW
    pltpu.sync_copy(verb_hbm.at[pl.ds(base, _BPW)], vidx)
    pltpu.sync_copy(noun_hbm.at[pl.ds(base, _BPW)], nidx)
    pltpu.sync_copy(color_hbm.at[pl.ds(base, _BPW)], cidx)
    for j in range(_DV):
        for c in range(_BPW // _L):
            sl = pl.ds(c * _L, _L)
            iv[j, sl] = vidx[sl] + j * _VV
            inn[j, sl] = nidx[sl] + j * _VN
    for j in range(_DC):
        for c in range(_BPW // _L):
            sl = pl.ds(c * _L, _L)
            ic[j, sl] = cidx[sl] + j * _VC
    descs = []
    for j in range(_DV):
        descs.append(pltpu.async_copy(evf_hbm.at[iv.at[j]], rv.at[j], sem_v))
        descs.append(pltpu.async_copy(enf_hbm.at[inn.at[j]], rn.at[j], sem_n))
    for j in range(_DC):
        descs.append(pltpu.async_copy(ecf_hbm.at[ic.at[j]], rc.at[j], sem_c))
    for d in descs:
        d.wait()
    for j in range(_DV):
        pltpu.sync_copy(rv.at[j], memb_out.at[j, pl.ds(base, _BPW)])
        pltpu.sync_copy(rn.at[j], memb_out.at[_DV + j, pl.ds(base, _BPW)])
    for j in range(_DC):
        pltpu.sync_copy(rc.at[j], memb_out.at[_DV + _DN + j, pl.ds(base, _BPW)])


def _sc_gather(verb, noun, color, evf, enf, ecf):
    mesh = plsc.VectorSubcoreMesh(core_axis_name="c", subcore_axis_name="s",
                                  num_cores=_NC, num_subcores=_NS)
    f = pl.kernel(
        _sc_gather_body,
        out_type=jax.ShapeDtypeStruct((_DM, _B), jnp.float32),
        mesh=mesh,
        scratch_types=[pltpu.VMEM((_BPW,), jnp.int32),
                       pltpu.VMEM((_BPW,), jnp.int32),
                       pltpu.VMEM((_BPW,), jnp.int32),
                       pltpu.VMEM((_DV, _BPW), jnp.int32),
                       pltpu.VMEM((_DN, _BPW), jnp.int32),
                       pltpu.VMEM((_DC, _BPW), jnp.int32),
                       pltpu.VMEM((_DV, _BPW), jnp.float32),
                       pltpu.VMEM((_DN, _BPW), jnp.float32),
                       pltpu.VMEM((_DC, _BPW), jnp.float32),
                       pltpu.SemaphoreType.DMA,
                       pltpu.SemaphoreType.DMA,
                       pltpu.SemaphoreType.DMA],
    )
    return f(verb, noun, color, evf, enf, ecf)


_TCB = 128


def _img_body(img_ref, out_ref):
    out_ref[...] = img_ref[...].reshape(_IMG_D, _TCB)


def _img_copy(img_t):
    return pl.pallas_call(
        _img_body,
        grid=(_B // _TCB,),
        in_specs=[pl.BlockSpec((3, 64, 64, _TCB), lambda i: (0, 0, 0, i))],
        out_specs=pl.BlockSpec((_IMG_D, _TCB), lambda i: (0, i)),
        out_shape=jax.ShapeDtypeStruct((_OUT_D, _B), jnp.float32),
    )(img_t)


def _memb_body(out_in, memb_ref, out_ref):
    out_ref[...] = memb_ref[...]


def _memb_write(out1, memb):
    nblk = _DM // 8
    return pl.pallas_call(
        _memb_body,
        grid=(nblk,),
        in_specs=[pl.BlockSpec((8, _B), lambda i: (_IMG_D // 8 + i, 0)),
                  pl.BlockSpec((8, _B), lambda i: (i, 0))],
        out_specs=pl.BlockSpec((8, _B), lambda i: (_IMG_D // 8 + i, 0)),
        out_shape=jax.ShapeDtypeStruct((_OUT_D, _B), jnp.float32),
        input_output_aliases={0: 0},
    )(out1, memb)


def kernel(image, verb, noun, color, emb_v, emb_n, emb_c):
    img_t = jnp.transpose(image.astype(jnp.float32), (1, 2, 3, 0))
    verb = verb.astype(jnp.int32)
    noun = noun.astype(jnp.int32)
    color = color.astype(jnp.int32)
    evf = emb_v.astype(jnp.float32).T.reshape(-1)
    enf = emb_n.astype(jnp.float32).T.reshape(-1)
    ecf = emb_c.astype(jnp.float32).T.reshape(-1)
    memb = _sc_gather(verb, noun, color, evf, enf, ecf)
    out_t = _img_copy(img_t)
    out_t = _memb_write(out_t, memb)
    return out_t.T


# speedup vs baseline: 3.9479x; 1.6348x over previous
import jax
import jax.numpy as jnp
from jax import lax
from jax.experimental import pallas as pl
from jax.experimental.pallas import tpu as pltpu
from jax.experimental.pallas import tpu_sc as plsc

_B = 4096
_IMG_D = 3 * 64 * 64
_DV, _DN, _DC = 16, 16, 8
_DM = _DV + _DN + _DC
_OUT_D = _IMG_D + _DM
_VV, _VN, _VC = 1000, 100000, 16

_NC, _NS = 2, 16
_NW = _NC * _NS
_BPW = _B // _NW
_L = 16


def _sc_gather_body(verb_hbm, noun_hbm, color_hbm, emb_v_hbm, emb_n_hbm, emb_c_hbm,
                    ev_out, en_out, ec_out,
                    vidx, nidx, cidx, rv, rn, rc, sem):
    wid = lax.axis_index("s") * _NC + lax.axis_index("c")
    base = wid * _BPW
    pltpu.sync_copy(verb_hbm.at[pl.ds(base, _BPW)], vidx)
    pltpu.sync_copy(noun_hbm.at[pl.ds(base, _BPW)], nidx)
    pltpu.sync_copy(color_hbm.at[pl.ds(base, _BPW)], cidx)
    descs = []
    for g in range(_BPW // _L):
        vv = vidx[pl.ds(g * _L, _L)]
        nv = nidx[pl.ds(g * _L, _L)]
        cv = cidx[pl.ds(g * _L, _L)]
        for i in range(_L):
            r = g * _L + i
            descs.append(pltpu.async_copy(emb_v_hbm.at[vv[i]], rv.at[r], sem))
            descs.append(pltpu.async_copy(emb_n_hbm.at[nv[i]], rn.at[r], sem))
            descs.append(pltpu.async_copy(emb_c_hbm.at[cv[i]], rc.at[r], sem))
    for d in descs:
        d.wait()
    pltpu.sync_copy(rv, ev_out.at[pl.ds(base, _BPW)])
    pltpu.sync_copy(rn, en_out.at[pl.ds(base, _BPW)])
    pltpu.sync_copy(rc, ec_out.at[pl.ds(base, _BPW)])


def _sc_gather(verb, noun, color, emb_v, emb_n, emb_c):
    mesh = plsc.VectorSubcoreMesh(core_axis_name="c", subcore_axis_name="s",
                                  num_cores=_NC, num_subcores=_NS)
    f = pl.kernel(
        _sc_gather_body,
        out_type=[jax.ShapeDtypeStruct((_B, _DV), jnp.float32),
                  jax.ShapeDtypeStruct((_B, _DN), jnp.float32),
                  jax.ShapeDtypeStruct((_B, _DC), jnp.float32)],
        mesh=mesh,
        scratch_types=[pltpu.VMEM((_BPW,), jnp.int32),
                       pltpu.VMEM((_BPW,), jnp.int32),
                       pltpu.VMEM((_BPW,), jnp.int32),
                       pltpu.VMEM((_BPW, _DV), jnp.float32),
                       pltpu.VMEM((_BPW, _DN), jnp.float32),
                       pltpu.VMEM((_BPW, _DC), jnp.float32),
                       pltpu.SemaphoreType.DMA],
    )
    return f(verb, noun, color, emb_v, emb_n, emb_c)


_HB = 8


def _img_body(img_ref, out_ref):
    out_ref[...] = img_ref[...].reshape(_HB * 64, _B)


def _img_copy(img_t):
    return pl.pallas_call(
        _img_body,
        grid=(3 * (64 // _HB),),
        in_specs=[pl.BlockSpec((1, _HB, 64, _B),
                               lambda i: (i // (64 // _HB), i % (64 // _HB), 0, 0))],
        out_specs=pl.BlockSpec((_HB * 64, _B), lambda i: (i, 0)),
        out_shape=jax.ShapeDtypeStruct((_OUT_D, _B), jnp.float32),
    )(img_t)


def _membt_body(out_in, e_ref, out_ref):
    out_ref[...] = e_ref[...].T


def _memb_write(out1, e, d, row0):
    return pl.pallas_call(
        _membt_body,
        grid=(1,),
        in_specs=[pl.BlockSpec((d, _B), lambda i: (row0 // d, 0)),
                  pl.BlockSpec((_B, d), lambda i: (0, 0))],
        out_specs=pl.BlockSpec((d, _B), lambda i: (row0 // d, 0)),
        out_shape=jax.ShapeDtypeStruct((_OUT_D, _B), jnp.float32),
        input_output_aliases={0: 0},
    )(out1, e)


def kernel(image, verb, noun, color, emb_v, emb_n, emb_c):
    img_t = jnp.transpose(image.astype(jnp.float32), (1, 2, 3, 0))
    verb = verb.astype(jnp.int32)
    noun = noun.astype(jnp.int32)
    color = color.astype(jnp.int32)
    ev, en, ec = _sc_gather(verb, noun, color,
                            emb_v.astype(jnp.float32),
                            emb_n.astype(jnp.float32),
                            emb_c.astype(jnp.float32))
    out_t = _img_copy(img_t)
    out_t = _memb_write(out_t, ev, 16, _IMG_D)
    out_t = _memb_write(out_t, en, 16, _IMG_D + 16)
    out_t = _memb_write(out_t, ec, 8, _IMG_D + 32)
    return out_t.T


# speedup vs baseline: 4.3813x; 1.1098x over previous
import jax
import jax.numpy as jnp
from jax import lax
from jax.experimental import pallas as pl
from jax.experimental.pallas import tpu as pltpu
from jax.experimental.pallas import tpu_sc as plsc

_B = 4096
_IMG_D = 3 * 64 * 64
_DV, _DN, _DC = 16, 16, 8
_DM = _DV + _DN + _DC
_OUT_D = _IMG_D + _DM
_VV, _VN, _VC = 1000, 100000, 16

_NC, _NS = 2, 16
_NW = _NC * _NS
_BPW = _B // _NW
_L = 16


def _sc_gather_body(verb_hbm, noun_hbm, color_hbm, emb_v_hbm, enf_hbm, emb_c_hbm,
                    ev_out, ent_out, ec_out,
                    vidx, nidx, cidx, rv, rc, ent, *rest):
    ibufs = rest[:_DN]
    sem, semn = rest[_DN:]
    wid = lax.axis_index("s") * _NC + lax.axis_index("c")
    base = wid * _BPW
    pltpu.sync_copy(verb_hbm.at[pl.ds(base, _BPW)], vidx)
    pltpu.sync_copy(noun_hbm.at[pl.ds(base, _BPW)], nidx)
    pltpu.sync_copy(color_hbm.at[pl.ds(base, _BPW)], cidx)
    for j in range(_DN):
        for c in range(_BPW // _L):
            sl = pl.ds(c * _L, _L)
            ibufs[j][sl] = nidx[sl] + j * _VN
    descs = []
    for j in range(_DN):
        descs.append(pltpu.async_copy(enf_hbm.at[ibufs[j]], ent.at[j], semn))
    for g in range(_BPW // _L):
        vv = vidx[pl.ds(g * _L, _L)]
        cv = cidx[pl.ds(g * _L, _L)]
        for i in range(_L):
            r = g * _L + i
            descs.append(pltpu.async_copy(emb_v_hbm.at[vv[i]], rv.at[r], sem))
            descs.append(pltpu.async_copy(emb_c_hbm.at[cv[i]], rc.at[r], sem))
    for d in descs:
        d.wait()
    pltpu.sync_copy(rv, ev_out.at[pl.ds(base, _BPW)])
    pltpu.sync_copy(rc, ec_out.at[pl.ds(base, _BPW)])
    for j in range(_DN):
        pltpu.sync_copy(ent.at[j], ent_out.at[j, pl.ds(base, _BPW)])


def _sc_gather(verb, noun, color, emb_v, enf, emb_c):
    mesh = plsc.VectorSubcoreMesh(core_axis_name="c", subcore_axis_name="s",
                                  num_cores=_NC, num_subcores=_NS)
    f = pl.kernel(
        _sc_gather_body,
        out_type=[jax.ShapeDtypeStruct((_B, _DV), jnp.float32),
                  jax.ShapeDtypeStruct((_DN, _B), jnp.float32),
                  jax.ShapeDtypeStruct((_B, _DC), jnp.float32)],
        mesh=mesh,
        scratch_types=([pltpu.VMEM((_BPW,), jnp.int32),
                        pltpu.VMEM((_BPW,), jnp.int32),
                        pltpu.VMEM((_BPW,), jnp.int32),
                        pltpu.VMEM((_BPW, _DV), jnp.float32),
                        pltpu.VMEM((_BPW, _DC), jnp.float32),
                        pltpu.VMEM((_DN, _BPW), jnp.float32)]
                       + [pltpu.VMEM((_BPW,), jnp.int32)] * _DN
                       + [pltpu.SemaphoreType.DMA,
                          pltpu.SemaphoreType.DMA]),
    )
    return f(verb, noun, color, emb_v, enf, emb_c)


_HB = 8


def _img_body(img_ref, out_ref):
    out_ref[...] = img_ref[...].reshape(_HB * 64, _B)


def _img_copy(img_t):
    return pl.pallas_call(
        _img_body,
        grid=(3 * (64 // _HB),),
        in_specs=[pl.BlockSpec((1, _HB, 64, _B),
                               lambda i: (i // (64 // _HB), i % (64 // _HB), 0, 0))],
        out_specs=pl.BlockSpec((_HB * 64, _B), lambda i: (i, 0)),
        out_shape=jax.ShapeDtypeStruct((_OUT_D, _B), jnp.float32),
    )(img_t)


def _membt_body(out_in, e_ref, out_ref):
    out_ref[...] = e_ref[...].T


def _membc_body(out_in, e_ref, out_ref):
    out_ref[...] = e_ref[...]


def _memb_write(out1, e, d, row0, transpose=True):
    in_shape = (_B, d) if transpose else (d, _B)
    return pl.pallas_call(
        _membt_body if transpose else _membc_body,
        grid=(1,),
        in_specs=[pl.BlockSpec((d, _B), lambda i: (row0 // d, 0)),
                  pl.BlockSpec(in_shape, lambda i: (0, 0))],
        out_specs=pl.BlockSpec((d, _B), lambda i: (row0 // d, 0)),
        out_shape=jax.ShapeDtypeStruct((_OUT_D, _B), jnp.float32),
        input_output_aliases={0: 0},
    )(out1, e)


def kernel(image, verb, noun, color, emb_v, emb_n, emb_c):
    img_t = jnp.transpose(image.astype(jnp.float32), (1, 2, 3, 0))
    verb = verb.astype(jnp.int32)
    noun = noun.astype(jnp.int32)
    color = color.astype(jnp.int32)
    enf = emb_n.astype(jnp.float32).T.reshape(-1)
    ev, ent, ec = _sc_gather(verb, noun, color,
                             emb_v.astype(jnp.float32), enf,
                             emb_c.astype(jnp.float32))
    out_t = _img_copy(img_t)
    out_t = _memb_write(out_t, ev, 16, _IMG_D)
    out_t = _memb_write(out_t, ent, 16, _IMG_D + 16, transpose=False)
    out_t = _memb_write(out_t, ec, 8, _IMG_D + 32)
    return out_t.T


# speedup vs baseline: 4.4605x; 1.0181x over previous
import jax
import jax.numpy as jnp
from jax import lax
from jax.experimental import pallas as pl
from jax.experimental.pallas import tpu as pltpu
from jax.experimental.pallas import tpu_sc as plsc

_B = 4096
_IMG_D = 3 * 64 * 64
_DV, _DN, _DC = 16, 16, 8
_DM = _DV + _DN + _DC
_OUT_D = _IMG_D + _DM
_VV, _VN, _VC = 1000, 100000, 16

_NC, _NS = 2, 16
_NW = _NC * _NS
_BPW = _B // _NW
_L = 16


def _sc_gather_body(verb_hbm, noun_hbm, color_hbm, emb_v_hbm, enf_hbm, emb_c_hbm,
                    ev_out, ent_out, ec_out,
                    vidx, nidx, cidx, rv, rc, ent, *rest):
    ibufs = rest[:_DN]
    sem, semn = rest[_DN:]
    wid = lax.axis_index("s") * _NC + lax.axis_index("c")
    base = wid * _BPW
    pltpu.sync_copy(verb_hbm.at[pl.ds(base, _BPW)], vidx)
    pltpu.sync_copy(noun_hbm.at[pl.ds(base, _BPW)], nidx)
    pltpu.sync_copy(color_hbm.at[pl.ds(base, _BPW)], cidx)
    for j in range(_DN):
        for c in range(_BPW // _L):
            sl = pl.ds(c * _L, _L)
            ibufs[j][sl] = nidx[sl] + j * _VN
    descs = []
    for j in range(_DN):
        descs.append(pltpu.async_copy(enf_hbm.at[ibufs[j]], ent.at[j], semn))
    for g in range(_BPW // _L):
        vv = vidx[pl.ds(g * _L, _L)]
        cv = cidx[pl.ds(g * _L, _L)]
        for i in range(_L):
            r = g * _L + i
            descs.append(pltpu.async_copy(emb_v_hbm.at[vv[i]], rv.at[r], sem))
            descs.append(pltpu.async_copy(emb_c_hbm.at[cv[i]], rc.at[r], sem))
    for d in descs:
        d.wait()
    pltpu.sync_copy(rv, ev_out.at[pl.ds(base, _BPW)])
    pltpu.sync_copy(rc, ec_out.at[pl.ds(base, _BPW)])
    for j in range(_DN):
        pltpu.sync_copy(ent.at[j], ent_out.at[j, pl.ds(base, _BPW)])


def _sc_gather(verb, noun, color, emb_v, enf, emb_c):
    mesh = plsc.VectorSubcoreMesh(core_axis_name="c", subcore_axis_name="s",
                                  num_cores=_NC, num_subcores=_NS)
    f = pl.kernel(
        _sc_gather_body,
        out_type=[jax.ShapeDtypeStruct((_B, _DV), jnp.float32),
                  jax.ShapeDtypeStruct((_DN, _B), jnp.float32),
                  jax.ShapeDtypeStruct((_B, _DC), jnp.float32)],
        mesh=mesh,
        scratch_types=([pltpu.VMEM((_BPW,), jnp.int32),
                        pltpu.VMEM((_BPW,), jnp.int32),
                        pltpu.VMEM((_BPW,), jnp.int32),
                        pltpu.VMEM((_BPW, _DV), jnp.float32),
                        pltpu.VMEM((_BPW, _DC), jnp.float32),
                        pltpu.VMEM((_DN, _BPW), jnp.float32)]
                       + [pltpu.VMEM((_BPW,), jnp.int32)] * _DN
                       + [pltpu.SemaphoreType.DMA,
                          pltpu.SemaphoreType.DMA]),
    )
    return f(verb, noun, color, emb_v, enf, emb_c)


_HB = 8


def _img_body(img_ref, out_ref):
    out_ref[...] = img_ref[...].reshape(_HB * 64, _B)


def _img_copy(img_t):
    return pl.pallas_call(
        _img_body,
        grid=(3 * (64 // _HB),),
        in_specs=[pl.BlockSpec((1, _HB, 64, _B),
                               lambda i: (i // (64 // _HB), i % (64 // _HB), 0, 0))],
        out_specs=pl.BlockSpec((_HB * 64, _B), lambda i: (i, 0)),
        out_shape=jax.ShapeDtypeStruct((_OUT_D, _B), jnp.float32),
    )(img_t)


def _memb_body(out_in, ev_ref, ent_ref, ec_ref, out_ref):
    out_ref[0:_DV, :] = ev_ref[...].T
    out_ref[_DV:_DV + _DN, :] = ent_ref[...]
    out_ref[_DV + _DN:_DM, :] = ec_ref[...].T
    out_ref[_DM:48, :] = jnp.zeros((48 - _DM, _B), jnp.float32)


def _memb_write(out1, ev, ent, ec):
    return pl.pallas_call(
        _memb_body,
        grid=(1,),
        in_specs=[pl.BlockSpec((48, _B), lambda i: (_IMG_D // 48, 0)),
                  pl.BlockSpec((_B, _DV), lambda i: (0, 0)),
                  pl.BlockSpec((_DN, _B), lambda i: (0, 0)),
                  pl.BlockSpec((_B, _DC), lambda i: (0, 0))],
        out_specs=pl.BlockSpec((48, _B), lambda i: (_IMG_D // 48, 0)),
        out_shape=jax.ShapeDtypeStruct((_OUT_D, _B), jnp.float32),
        input_output_aliases={0: 0},
    )(out1, ev, ent, ec)


def kernel(image, verb, noun, color, emb_v, emb_n, emb_c):
    img_t = jnp.transpose(image.astype(jnp.float32), (1, 2, 3, 0))
    verb = verb.astype(jnp.int32)
    noun = noun.astype(jnp.int32)
    color = color.astype(jnp.int32)
    enf = emb_n.astype(jnp.float32).T.reshape(-1)
    ev, ent, ec = _sc_gather(verb, noun, color,
                             emb_v.astype(jnp.float32), enf,
                             emb_c.astype(jnp.float32))
    out_t = _img_copy(img_t)
    out_t = _memb_write(out_t, ev, ent, ec)
    return out_t.T
